# Initial kernel scaffold; baseline (speedup 1.0000x reference)
import functools

import jax
import jax.numpy as jnp
from jax import lax
from jax.experimental import pallas as pl
from jax.experimental.pallas import tpu as pltpu
from jax.experimental.pallas import tpu_sc as plsc

_NUM_CLASSES = 10
_CONF_PER_CLASS = 1000
_D = 32 * 32 * 3
_B = 1024
_N = 10
_BN = _B * _N

_NC, _NS, _L = 2, 16, 16
_NW = _NC * _NS
_ROWS_PER_W = _BN // _NW
_CH = 16
_NCHUNK = _ROWS_PER_W // _CH
_BATCH_PER_W = _B // _NW


@functools.partial(
    pl.kernel,
    out_type=jax.ShapeDtypeStruct((_BN, _D), jnp.float32),
    mesh=plsc.VectorSubcoreMesh(core_axis_name="c", subcore_axis_name="s"),
    scratch_types=[
        pltpu.VMEM((_BATCH_PER_W,), jnp.int32),
        pltpu.VMEM((_ROWS_PER_W,), jnp.int32),
        pltpu.VMEM((_ROWS_PER_W,), jnp.int32),
        pltpu.VMEM((_CH, _D), jnp.float32),
        pltpu.VMEM((_CH, _D), jnp.float32),
        pltpu.SemaphoreType.DMA,
        pltpu.SemaphoreType.DMA,
        pltpu.SemaphoreType.DMA,
        pltpu.SemaphoreType.DMA,
    ],
)
def _gather_rows(table, label, sidx, out, lab_v, sid_v, idx_v,
                 buf0, buf1, gs0, gs1, os0, os1):
    wid = lax.axis_index("s") * _NC + lax.axis_index("c")
    base = wid * _ROWS_PER_W

    pltpu.sync_copy(label.at[pl.ds(wid * _BATCH_PER_W, _BATCH_PER_W)], lab_v)
    pltpu.sync_copy(sidx.at[pl.ds(base, _ROWS_PER_W)], sid_v)

    lane = lax.iota(jnp.int32, _L)
    for v in range(_ROWS_PER_W // _L):
        pos = lane + (v * _L)
        j = pos // _N
        labs = plsc.load_gather(lab_v, [j])
        idx_v[pl.ds(v * _L, _L)] = labs * _CONF_PER_CLASS + sid_v[pl.ds(v * _L, _L)]

    bufs = (buf0, buf1)
    gsems = (gs0, gs1)
    osems = (os0, os1)
    cin = [None] * _NCHUNK
    cout = [None] * _NCHUNK
    for g in range(_NCHUNK):
        b = g & 1
        if g >= 2:
            cout[g - 2].wait()
        iv = idx_v[pl.ds(g * _CH, _CH)]
        cin[g] = pltpu.async_copy(table.at[iv], bufs[b], gsems[b])
        if g >= 1:
            cin[g - 1].wait()
            cout[g - 1] = pltpu.async_copy(
                bufs[1 - b], out.at[pl.ds(base + (g - 1) * _CH, _CH)],
                osems[1 - b])
    last = _NCHUNK - 1
    cin[last].wait()
    cout[last] = pltpu.async_copy(
        bufs[last & 1], out.at[pl.ds(base + last * _CH, _CH)], osems[last & 1])
    cout[last - 1].wait()
    cout[last].wait()


def kernel(x_s, label, sample_idx, confounder_queue):
    del x_s
    table = confounder_queue.reshape(_NUM_CLASSES * _CONF_PER_CLASS, _D)
    sidx = sample_idx.reshape(_BN).astype(jnp.int32)
    out = _gather_rows(table, label.astype(jnp.int32), sidx)
    return out.reshape(_B, _N, 32, 32, 3)


# speedup vs baseline: 2.3728x; 2.3728x over previous
import functools

import jax
import jax.numpy as jnp
from jax import lax
from jax.experimental import pallas as pl
from jax.experimental.pallas import tpu as pltpu
from jax.experimental.pallas import tpu_sc as plsc

_NUM_CLASSES = 10
_CONF_PER_CLASS = 1000
_D = 32 * 32 * 3
_B = 1024
_N = 10
_BN = _B * _N

_NC, _NS, _L = 2, 16, 16
_NW = _NC * _NS
_ROWS_PER_W = _BN // _NW
_CH = 16
_NCHUNK = _ROWS_PER_W // _CH


def _flat_idx_body(label_ref, sidx_ref, out_ref):
    out_ref[...] = label_ref[...] * _CONF_PER_CLASS + sidx_ref[...]


def _flat_idx(label, sample_idx):
    return pl.pallas_call(
        _flat_idx_body,
        out_shape=jax.ShapeDtypeStruct((_B, _N), jnp.int32),
    )(jnp.broadcast_to(label[:, None], (_B, _N)), sample_idx)


@functools.partial(
    pl.kernel,
    out_type=jax.ShapeDtypeStruct((_BN, _D), jnp.float32),
    mesh=plsc.VectorSubcoreMesh(core_axis_name="c", subcore_axis_name="s"),
    scratch_types=[
        pltpu.VMEM((_ROWS_PER_W,), jnp.int32),
        pltpu.VMEM((_CH, _D), jnp.float32),
        pltpu.VMEM((_CH, _D), jnp.float32),
        pltpu.SemaphoreType.DMA,
        pltpu.SemaphoreType.DMA,
        pltpu.SemaphoreType.DMA,
        pltpu.SemaphoreType.DMA,
    ],
)
def _gather_rows(table, fidx, out, idx_v, buf0, buf1, gs0, gs1, os0, os1):
    wid = lax.axis_index("s") * _NC + lax.axis_index("c")
    base = wid * _ROWS_PER_W

    pltpu.sync_copy(fidx.at[pl.ds(base, _ROWS_PER_W)], idx_v)

    bufs = (buf0, buf1)
    gsems = (gs0, gs1)
    osems = (os0, os1)
    cin = [None] * _NCHUNK
    cout = [None] * _NCHUNK
    for g in range(_NCHUNK):
        b = g & 1
        if g >= 2:
            cout[g - 2].wait()
        iv = idx_v[pl.ds(g * _CH, _CH)]
        cin[g] = pltpu.async_copy(table.at[iv], bufs[b], gsems[b])
        if g >= 1:
            cin[g - 1].wait()
            cout[g - 1] = pltpu.async_copy(
                bufs[1 - b], out.at[pl.ds(base + (g - 1) * _CH, _CH)],
                osems[1 - b])
    last = _NCHUNK - 1
    cin[last].wait()
    cout[last] = pltpu.async_copy(
        bufs[last & 1], out.at[pl.ds(base + last * _CH, _CH)], osems[last & 1])
    cout[last - 1].wait()
    cout[last].wait()


def kernel(x_s, label, sample_idx, confounder_queue):
    del x_s
    table = confounder_queue.reshape(_NUM_CLASSES * _CONF_PER_CLASS, _D)
    fidx = _flat_idx(label.astype(jnp.int32), sample_idx.astype(jnp.int32))
    out = _gather_rows(table, fidx.reshape(_BN))
    return out.reshape(_B, _N, 32, 32, 3)


# speedup vs baseline: 5.4304x; 2.2886x over previous
import jax
import jax.numpy as jnp
import numpy as np
from jax import lax
from jax.experimental import pallas as pl
from jax.experimental.pallas import tpu as pltpu
from jax.experimental.pallas import tpu_sc as plsc

_NUM_CLASSES = 10
_CONF_PER_CLASS = 1000
_D = 32 * 32 * 3
_B = 1024
_N = 10
_BN = _B * _N

_NC, _NS, _L = 2, 16, 16
_NW = _NC * _NS
_BATCH_PER_W = _B // _NW
_IDX_PER_W = _BATCH_PER_W * _N


def _flat_idx_body(label_ref, sidx_ref, out_ref):
    out_ref[...] = label_ref[...] * _CONF_PER_CLASS + sidx_ref[...]


def _flat_idx(label, sample_idx):
    return pl.pallas_call(
        _flat_idx_body,
        out_shape=jax.ShapeDtypeStruct((_B, _N), jnp.int32),
    )(jnp.broadcast_to(label[:, None], (_B, _N)), sample_idx)


_SCRATCH = [
    pltpu.VMEM((_IDX_PER_W,), jnp.int32),
    pltpu.VMEM((_L, _D), jnp.float32),
    pltpu.VMEM((_L, _D), jnp.float32),
    pltpu.SemaphoreType.DMA,
    pltpu.SemaphoreType.DMA,
    pltpu.SemaphoreType.DMA,
    pltpu.SemaphoreType.DMA,
]


def _gather_body(table, fidx, out, idx_v, buf0, buf1, gs0, gs1, os0, os1):
    wid = lax.axis_index("s") * _NC + lax.axis_index("c")
    jbase = wid * _BATCH_PER_W

    pltpu.sync_copy(fidx.at[pl.ds(jbase * _N, _IDX_PER_W)], idx_v)

    def elem_iv(g):
        base16 = (g * _N) // _L * _L
        off = g * _N - base16
        lane = lax.iota(jnp.int32, _L)
        pick = jnp.minimum(lane + off, off + _N - 1)
        w0 = idx_v[pl.ds(base16, _L)]
        lo = w0.at[jnp.minimum(pick, _L - 1)].get(mode="promise_in_bounds")
        if off + _N - 1 < _L:
            return lo
        w1 = idx_v[pl.ds(base16 + _L, _L)]
        hi = w1.at[jnp.maximum(pick - _L, 0)].get(mode="promise_in_bounds")
        return jnp.where(pick < _L, lo, hi)

    bufs = (buf0, buf1)
    gsems = (gs0, gs1)
    osems = (os0, os1)
    nch = _BATCH_PER_W
    cin = [None] * nch
    cout = [None] * nch
    for g in range(nch):
        b = g & 1
        if g >= 2:
            cout[g - 2].wait()
        cin[g] = pltpu.async_copy(table.at[elem_iv(g)], bufs[b], gsems[b])
        if g >= 1:
            cin[g - 1].wait()
            cout[g - 1] = pltpu.async_copy(
                bufs[1 - b].at[pl.ds(0, _N)], out.at[jbase + (g - 1)],
                osems[1 - b])
    last = nch - 1
    cin[last].wait()
    cout[last] = pltpu.async_copy(
        bufs[last & 1].at[pl.ds(0, _N)], out.at[jbase + last], osems[last & 1])
    cout[last - 1].wait()
    cout[last].wait()


_gather_rows = pl.kernel(
    _gather_body,
    out_type=jax.ShapeDtypeStruct((_B, _N, _D), jnp.float32),
    mesh=plsc.VectorSubcoreMesh(core_axis_name="c", subcore_axis_name="s"),
    compiler_params=pltpu.CompilerParams(use_tc_tiling_on_sc=False),
    scratch_types=_SCRATCH,
)


def kernel(x_s, label, sample_idx, confounder_queue):
    del x_s
    table = confounder_queue.reshape(_NUM_CLASSES * _CONF_PER_CLASS, _D)
    fidx = _flat_idx(label.astype(jnp.int32), sample_idx.astype(jnp.int32))
    out = _gather_rows(table, fidx.reshape(_BN))
    return out.reshape(_B, _N, 32, 32, 3)


# speedup vs baseline: 6.6416x; 1.2230x over previous
import jax
import jax.numpy as jnp
from jax import lax
from jax.experimental import pallas as pl
from jax.experimental.pallas import tpu as pltpu
from jax.experimental.pallas import tpu_sc as plsc

_NUM_CLASSES = 10
_CONF_PER_CLASS = 1000
_V = _NUM_CLASSES * _CONF_PER_CLASS
_H, _W, _C = 32, 32, 3
_D = _H * _W * _C
_B = 1024
_N = 10

_NC, _NS, _L = 2, 16, 16
_NW = _NC * _NS
_DPW = _D // _NW


def _flat_idx_body(label_ref, sidx_ref, out_ref):
    out_ref[...] = label_ref[...] * _CONF_PER_CLASS + sidx_ref[...]


def _flat_idx_t(label, sample_idx):
    return pl.pallas_call(
        _flat_idx_body,
        out_shape=jax.ShapeDtypeStruct((_N, _B), jnp.int32),
    )(jnp.broadcast_to(label[None, :], (_N, _B)), sample_idx.T)


_SCRATCH = [
    pltpu.VMEM((_N, _B), jnp.int32),
    pltpu.VMEM((_V,), jnp.float32),
    pltpu.VMEM((_V,), jnp.float32),
    pltpu.VMEM((_N, _B), jnp.float32),
    pltpu.VMEM((_N, _B), jnp.float32),
    pltpu.SemaphoreType.DMA,
    pltpu.SemaphoreType.DMA,
    pltpu.SemaphoreType.DMA,
    pltpu.SemaphoreType.DMA,
]


def _lut_row(wid, k):
    ch = k // _W
    w = k % _W
    return wid * _DPW + w * _C + ch


def _gather_body(tableT, fidxT, out, idx_v, lut0, lut1, ost0, ost1,
                 ls0, ls1, os0, os1):
    wid = lax.axis_index("s") * _NC + lax.axis_index("c")

    pltpu.sync_copy(fidxT, idx_v)

    luts = (lut0, lut1)
    osts = (ost0, ost1)
    lsems = (ls0, ls1)
    osems = (os0, os1)

    pltpu.async_copy(tableT.at[_lut_row(wid, 0)], lut0, ls0)

    @pl.loop(0, _DPW, step=2)
    def _(k0):
        for p in range(2):
            k = k0 + p

            @pl.when(k < _DPW - 1)
            def _():
                pltpu.make_async_copy(
                    tableT.at[_lut_row(wid, k + 1)], luts[1 - p],
                    lsems[1 - p]).start()

            pltpu.make_async_copy(
                tableT.at[_lut_row(wid, k)], luts[p], lsems[p]).wait()

            @pl.when(k >= 2)
            def _():
                for n in range(_N):
                    pltpu.make_async_copy(
                        osts[p].at[n], out.at[n, wid, 0, 0],
                        osems[p]).wait()

            for n in range(_N):
                for b0 in range(0, _B, _L):
                    iv = idx_v[n, pl.ds(b0, _L)]
                    osts[p][n, pl.ds(b0, _L)] = plsc.load_gather(
                        luts[p], [iv])

            ch = k // _W
            w = k % _W
            for n in range(_N):
                pltpu.async_copy(osts[p].at[n], out.at[n, wid, ch, w],
                                 osems[p])

    for p in range(2):
        for n in range(_N):
            pltpu.make_async_copy(
                osts[p].at[n], out.at[n, wid, 0, 0], osems[p]).wait()


_gather_rows = pl.kernel(
    _gather_body,
    out_type=jax.ShapeDtypeStruct((_N, _H, _C, _W, _B), jnp.float32),
    mesh=plsc.VectorSubcoreMesh(core_axis_name="c", subcore_axis_name="s"),
    compiler_params=pltpu.CompilerParams(
        use_tc_tiling_on_sc=False, needs_layout_passes=False),
    scratch_types=_SCRATCH,
)


def kernel(x_s, label, sample_idx, confounder_queue):
    del x_s
    tableT = confounder_queue.reshape(_V, _D).T
    fidxT = _flat_idx_t(label.astype(jnp.int32), sample_idx.astype(jnp.int32))
    out = _gather_rows(tableT, fidxT)
    return jnp.transpose(out, (4, 0, 1, 3, 2))


# speedup vs baseline: 7.7301x; 1.1639x over previous
import jax
import jax.numpy as jnp
from jax import lax
from jax.experimental import pallas as pl
from jax.experimental.pallas import tpu as pltpu
from jax.experimental.pallas import tpu_sc as plsc

_NUM_CLASSES = 10
_CONF_PER_CLASS = 1000
_V = _NUM_CLASSES * _CONF_PER_CLASS
_H, _W, _C = 32, 32, 3
_D = _H * _W * _C
_B = 1024
_N = 10

_NC, _NS, _L = 2, 16, 16
_NW = _NC * _NS
_DPW = _D // _NW


def _flat_idx_body(label_ref, sidx_ref, out_ref):
    out_ref[...] = label_ref[...] * _CONF_PER_CLASS + sidx_ref[...]


def _flat_idx_t(label, sample_idx):
    return pl.pallas_call(
        _flat_idx_body,
        out_shape=jax.ShapeDtypeStruct((_N, _B), jnp.int32),
    )(jnp.broadcast_to(label[None, :], (_N, _B)), sample_idx.T)


_SCRATCH = [
    pltpu.VMEM((_N, _B), jnp.int32),
    pltpu.VMEM((_V,), jnp.float32),
    pltpu.VMEM((_V,), jnp.float32),
    pltpu.VMEM((_N, 8, 128), jnp.float32),
    pltpu.VMEM((_N, 8, 128), jnp.float32),
    pltpu.SemaphoreType.DMA,
    pltpu.SemaphoreType.DMA,
    pltpu.SemaphoreType.DMA,
    pltpu.SemaphoreType.DMA,
]


def _lut_row(wid, k):
    ch = k // _W
    w = k % _W
    return wid * _DPW + w * _C + ch


def _gather_body(tableT, fidxT, out, idx_v, lut0, lut1, ost0, ost1,
                 ls0, ls1, os0, os1):
    wid = lax.axis_index("s") * _NC + lax.axis_index("c")

    pltpu.sync_copy(fidxT, idx_v)

    luts = (lut0, lut1)
    osts = (ost0, ost1)
    lsems = (ls0, ls1)
    osems = (os0, os1)

    def out_slab(n, k):
        ch = k // _W
        w = k % _W
        return out.at[n, wid, ch, w // 8, :, w % 8, :]

    pltpu.async_copy(tableT.at[_lut_row(wid, 0)], lut0, ls0)

    @pl.loop(0, _DPW, step=2)
    def _(k0):
        for p in range(2):
            k = k0 + p

            @pl.when(k < _DPW - 1)
            def _():
                pltpu.make_async_copy(
                    tableT.at[_lut_row(wid, k + 1)], luts[1 - p],
                    lsems[1 - p]).start()

            pltpu.make_async_copy(
                tableT.at[_lut_row(wid, k)], luts[p], lsems[p]).wait()

            @pl.when(k >= 2)
            def _():
                for n in range(_N):
                    pltpu.make_async_copy(
                        osts[p].at[n], out_slab(n, k), osems[p]).wait()

            for n in range(_N):
                for b0 in range(0, _B, _L):
                    iv = idx_v[n, pl.ds(b0, _L)]
                    osts[p][n, b0 // 128, pl.ds(b0 % 128, _L)] = (
                        plsc.load_gather(luts[p], [iv]))

            for n in range(_N):
                pltpu.async_copy(osts[p].at[n], out_slab(n, k), osems[p])

    for p in range(2):
        for n in range(_N):
            pltpu.make_async_copy(
                osts[p].at[n], out.at[n, wid, 0, 0, :, 0, :],
                osems[p]).wait()


_gather_rows = pl.kernel(
    _gather_body,
    out_type=jax.ShapeDtypeStruct((_N, _H, _C, _W // 8, 8, 8, 128),
                                  jnp.float32),
    mesh=plsc.VectorSubcoreMesh(core_axis_name="c", subcore_axis_name="s"),
    compiler_params=pltpu.CompilerParams(
        use_tc_tiling_on_sc=False, needs_layout_passes=False),
    scratch_types=_SCRATCH,
)


def kernel(x_s, label, sample_idx, confounder_queue):
    del x_s
    tableT = confounder_queue.reshape(_V, _D).T
    fidxT = _flat_idx_t(label.astype(jnp.int32), sample_idx.astype(jnp.int32))
    out7 = _gather_rows(tableT, fidxT)
    res = jnp.transpose(out7, (4, 6, 0, 1, 3, 5, 2))
    return res.reshape(_B, _N, _H, _W, _C)


# speedup vs baseline: 7.7723x; 1.0055x over previous
import jax
import jax.numpy as jnp
from jax import lax
from jax.experimental import pallas as pl
from jax.experimental.pallas import tpu as pltpu
from jax.experimental.pallas import tpu_sc as plsc

_NUM_CLASSES = 10
_CONF_PER_CLASS = 1000
_V = _NUM_CLASSES * _CONF_PER_CLASS
_H, _W, _C = 32, 32, 3
_D = _H * _W * _C
_B = 1024
_N = 10

_NC, _NS, _L = 2, 16, 16
_NW = _NC * _NS
_DPW = _D // _NW


def _flat_idx_body(label_ref, sidx_ref, out_ref):
    out_ref[...] = label_ref[...] * _CONF_PER_CLASS + sidx_ref[...]


def _flat_idx_t(label, sample_idx):
    return pl.pallas_call(
        _flat_idx_body,
        out_shape=jax.ShapeDtypeStruct((_N, _B), jnp.int32),
    )(jnp.broadcast_to(label[None, :], (_N, _B)), sample_idx.T)


_SCRATCH = (
    [pltpu.VMEM((_N, _B), jnp.int32)]
    + [pltpu.VMEM((_V,), jnp.float32) for _ in range(4)]
    + [pltpu.VMEM((_N, 8, 128), jnp.float32) for _ in range(2)]
    + [pltpu.SemaphoreType.DMA for _ in range(4)]
    + [pltpu.SemaphoreType.DMA for _ in range(2)]
)


def _lut_row(wid, k):
    ch = k // _W
    w = k % _W
    return wid * _DPW + w * _C + ch


def _gather_body(tableT, fidxT, out, idx_v, lut0, lut1, lut2, lut3,
                 ost0, ost1, ls0, ls1, ls2, ls3, os0, os1):
    wid = lax.axis_index("s") * _NC + lax.axis_index("c")

    pltpu.sync_copy(fidxT, idx_v)

    luts = (lut0, lut1, lut2, lut3)
    osts = (ost0, ost1)
    lsems = (ls0, ls1, ls2, ls3)
    osems = (os0, os1)

    def out_slab(n, k):
        ch = k // _W
        w = k % _W
        return out.at[n, wid, ch, w // 8, :, w % 8, :]

    for q in range(4):
        pltpu.async_copy(tableT.at[_lut_row(wid, q)], luts[q], lsems[q])

    @pl.loop(0, _DPW, step=4)
    def _(k0):
        for p2 in range(2):
            kA = k0 + 2 * p2
            bA, bB = 2 * p2, 2 * p2 + 1

            pltpu.make_async_copy(
                tableT.at[_lut_row(wid, kA)], luts[bA], lsems[bA]).wait()
            pltpu.make_async_copy(
                tableT.at[_lut_row(wid, kA)], luts[bB], lsems[bB]).wait()

            @pl.when(kA >= 2)
            def _():
                for n in range(_N):
                    pltpu.make_async_copy(
                        osts[0].at[n], out_slab(n, kA), osems[0]).wait()
                    pltpu.make_async_copy(
                        osts[1].at[n], out_slab(n, kA), osems[1]).wait()

            for n in range(_N):
                for b0 in range(0, _B, _L):
                    iv = idx_v[n, pl.ds(b0, _L)]
                    osts[0][n, b0 // 128, pl.ds(b0 % 128, _L)] = (
                        plsc.load_gather(luts[bA], [iv]))
                    osts[1][n, b0 // 128, pl.ds(b0 % 128, _L)] = (
                        plsc.load_gather(luts[bB], [iv]))

            for n in range(_N):
                pltpu.async_copy(osts[0].at[n], out_slab(n, kA), osems[0])
                pltpu.async_copy(osts[1].at[n], out_slab(n, kA + 1),
                                 osems[1])

            @pl.when(kA + 4 < _DPW)
            def _():
                pltpu.make_async_copy(
                    tableT.at[_lut_row(wid, kA + 4)], luts[bA],
                    lsems[bA]).start()
                pltpu.make_async_copy(
                    tableT.at[_lut_row(wid, kA + 5)], luts[bB],
                    lsems[bB]).start()

    for s in range(2):
        for n in range(_N):
            pltpu.make_async_copy(
                osts[s].at[n], out.at[n, wid, 0, 0, :, 0, :],
                osems[s]).wait()


_gather_rows = pl.kernel(
    _gather_body,
    out_type=jax.ShapeDtypeStruct((_N, _H, _C, _W // 8, 8, 8, 128),
                                  jnp.float32),
    mesh=plsc.VectorSubcoreMesh(core_axis_name="c", subcore_axis_name="s"),
    compiler_params=pltpu.CompilerParams(
        use_tc_tiling_on_sc=False, needs_layout_passes=False),
    scratch_types=_SCRATCH,
)


def kernel(x_s, label, sample_idx, confounder_queue):
    del x_s
    tableT = confounder_queue.reshape(_V, _D).T
    fidxT = _flat_idx_t(label.astype(jnp.int32), sample_idx.astype(jnp.int32))
    out7 = _gather_rows(tableT, fidxT)
    res = jnp.transpose(out7, (4, 6, 0, 1, 3, 5, 2))
    return res.reshape(_B, _N, _H, _W, _C)


# speedup vs baseline: 9.4188x; 1.2118x over previous
import jax
import jax.numpy as jnp
from jax import lax
from jax.experimental import pallas as pl
from jax.experimental.pallas import tpu as pltpu
from jax.experimental.pallas import tpu_sc as plsc

_NUM_CLASSES = 10
_CONF_PER_CLASS = 1000
_V = _NUM_CLASSES * _CONF_PER_CLASS
_H, _W, _C = 32, 32, 3
_D = _H * _W * _C
_B = 1024
_N = 10

_NC, _NS, _L = 2, 16, 16
_NW = _NC * _NS
_DPW = _D // _NW


def _flat_idx_body(label_ref, sidx_ref, out_ref):
    out_ref[...] = label_ref[...] * _CONF_PER_CLASS + sidx_ref[...]


def _flat_idx_t(label, sample_idx):
    return pl.pallas_call(
        _flat_idx_body,
        out_shape=jax.ShapeDtypeStruct((_N, _B), jnp.int32),
    )(jnp.broadcast_to(label[None, :], (_N, _B)), sample_idx.T)


_SCRATCH = (
    [pltpu.VMEM((_N, _B), jnp.int32)]
    + [pltpu.VMEM((_V,), jnp.float32) for _ in range(4)]
    + [pltpu.VMEM((_N, 8, 128), jnp.float32) for _ in range(2)]
    + [pltpu.SemaphoreType.DMA for _ in range(4)]
    + [pltpu.SemaphoreType.DMA for _ in range(2)]
)


def _lut_row(wid, k):
    ch = k // _W
    w = k % _W
    return wid * _DPW + w * _C + ch


def _gather_body(tableT, fidxT, out, idx_v, lut0, lut1, lut2, lut3,
                 ost0, ost1, ls0, ls1, ls2, ls3, os0, os1):
    wid = lax.axis_index("s") * _NC + lax.axis_index("c")

    pltpu.sync_copy(fidxT, idx_v)

    luts = (lut0, lut1, lut2, lut3)
    osts = (ost0, ost1)
    lsems = (ls0, ls1, ls2, ls3)
    osems = (os0, os1)

    def out_slab(n, k):
        ch = k // _W
        w = k % _W
        return out.at[n, wid, ch, w // 8, :, w % 8, :]

    for q in range(4):
        pltpu.async_copy(tableT.at[_lut_row(wid, q)], luts[q], lsems[q])

    @pl.loop(0, _DPW, step=4)
    def _(k0):
        for p2 in range(2):
            kA = k0 + 2 * p2
            bA, bB = 2 * p2, 2 * p2 + 1

            pltpu.make_async_copy(
                tableT.at[_lut_row(wid, kA)], luts[bA], lsems[bA]).wait()
            pltpu.make_async_copy(
                tableT.at[_lut_row(wid, kA)], luts[bB], lsems[bB]).wait()

            @pl.when(kA >= 2)
            def _():
                for n in range(_N):
                    pltpu.make_async_copy(
                        osts[0].at[n], out_slab(n, kA), osems[0]).wait()
                    pltpu.make_async_copy(
                        osts[1].at[n], out_slab(n, kA), osems[1]).wait()

            for n in range(_N):
                for b0 in range(0, _B, _L):
                    iv = idx_v[n, pl.ds(b0, _L)]
                    osts[0][n, b0 // 128, pl.ds(b0 % 128, _L)] = (
                        plsc.load_gather(luts[bA], [iv]))
                    osts[1][n, b0 // 128, pl.ds(b0 % 128, _L)] = (
                        plsc.load_gather(luts[bB], [iv]))

            for n in range(_N):
                pltpu.async_copy(osts[0].at[n], out_slab(n, kA), osems[0])
                pltpu.async_copy(osts[1].at[n], out_slab(n, kA + 1),
                                 osems[1])

            @pl.when(kA + 4 < _DPW)
            def _():
                pltpu.make_async_copy(
                    tableT.at[_lut_row(wid, kA + 4)], luts[bA],
                    lsems[bA]).start()
                pltpu.make_async_copy(
                    tableT.at[_lut_row(wid, kA + 5)], luts[bB],
                    lsems[bB]).start()

    for s in range(2):
        for n in range(_N):
            pltpu.make_async_copy(
                osts[s].at[n], out.at[n, wid, 0, 0, :, 0, :],
                osems[s]).wait()


_gather_rows = pl.kernel(
    _gather_body,
    out_type=jax.ShapeDtypeStruct((_N, _H, _C, _W // 8, 8, 8, 128),
                                  jnp.float32),
    mesh=plsc.VectorSubcoreMesh(core_axis_name="c", subcore_axis_name="s"),
    compiler_params=pltpu.CompilerParams(
        use_tc_tiling_on_sc=False, needs_layout_passes=False),
    scratch_types=_SCRATCH,
)


def kernel(x_s, label, sample_idx, confounder_queue):
    del x_s
    tableT = confounder_queue.transpose(2, 3, 4, 0, 1).reshape(_D, _V)
    fidxT = _flat_idx_t(label.astype(jnp.int32), sample_idx.astype(jnp.int32))
    out7 = _gather_rows(tableT, fidxT)
    res = jnp.transpose(out7, (4, 6, 0, 1, 3, 5, 2))
    return res.reshape(_B, _N, _H, _W, _C)
